# Initial kernel scaffold; baseline (speedup 1.0000x reference)
import jax
import jax.numpy as jnp
from jax.experimental import pallas as pl


def kernel(x, edge_index, W_self_0, W_neigh_0, b_0, W_self_1, W_neigh_1, b_1, W_self_2, W_neigh_2, b_2):
    raise NotImplementedError("write your pallas kernel here")


# speedup vs baseline: 12.4026x; 12.4026x over previous
import functools

import jax
import jax.numpy as jnp
from jax import lax
from jax.experimental import pallas as pl
from jax.experimental.pallas import tpu as pltpu
from jax.experimental.pallas import tpu_sc as plsc

N = 10000
E = 320000
F = 128
HID = 128
C = 47
CP = 128

NC = 2
NS = 16
NW = NC * NS
EPT = E // NW
CH = 80
NCHUNK = EPT // CH
NP = 10240
RPT = NP // NS
NB = 4


def _drain_counts():
  issued = [0] * NB
  waited = [0] * NB
  for i in range(NCHUNK):
    issued[i % NB] += 1
    if i >= 2:
      waited[(i + 2) % NB] += 1
  return [issued[j] - waited[j] for j in range(NB)]


def _make_spmm(D):
  mesh = plsc.VectorSubcoreMesh(core_axis_name="c", subcore_axis_name="s",
                                num_cores=NC, num_subcores=NS)
  out_type = jax.ShapeDtypeStruct((NC, NP, D), jnp.float32)
  scratch = [
      pltpu.VMEM((NB, 2, CH), jnp.int32),
      pltpu.VMEM((NB, CH, D), jnp.float32),
      pltpu.VMEM_SHARED((NP, D), jnp.float32),
  ] + [pltpu.SemaphoreType.DMA] * (3 * NB)
  KR = RPT // CH

  def body(ei_hbm, g_hbm, pagg_hbm, idx_v, rows_v, agg_s, *sems):
    stage_v = rows_v.at[0]
    lsem = sems[0:NB]
    gsem = sems[NB:2 * NB]
    ssem = sems[2 * NB:3 * NB]
    c = lax.axis_index("c")
    s = lax.axis_index("s")
    zv = jnp.zeros((16,), jnp.float32)
    def zrow(i, carry):
      for j in range(D // 16):
        stage_v[i, pl.ds(j * 16, 16)] = zv
      return carry
    lax.fori_loop(0, CH, zrow, 0)
    for k in range(KR):
      pltpu.sync_copy(stage_v, agg_s.at[pl.ds(s * RPT + k * CH, CH)])
    plsc.subcore_barrier()

    base = (s * NC + c) * NCHUNK

    def load(i, b):
      pltpu.async_copy(ei_hbm.at[:, base + i, :], idx_v.at[b], lsem[b])

    def wait_load(b):
      pltpu.make_async_copy(ei_hbm.at[:, base, :], idx_v.at[b], lsem[b]).wait()

    def wait_gather(b):
      pltpu.make_async_copy(g_hbm.at[pl.ds(0, CH)], rows_v.at[b], gsem[b]).wait()

    def wait_scatter(b):
      pltpu.make_async_copy(g_hbm.at[pl.ds(0, CH)], rows_v.at[b], ssem[b]).wait()

    def emit(i, b, wait_s, do_next, guard_load):
      bn = (b + 1) % NB
      bn2 = (b + 2) % NB
      if wait_s:
        wait_scatter(bn2)
      if do_next:
        wait_load(bn)
        pltpu.async_copy(g_hbm.at[idx_v.at[bn, 0]], rows_v.at[bn], gsem[bn])
        if guard_load:
          @pl.when(i + 2 < NCHUNK)
          def _():
            load(i + 2, bn2)
        else:
          load(i + 2, bn2)
      wait_gather(b)
      pltpu.async_copy(rows_v.at[b], agg_s.at[idx_v.at[b, 1]], ssem[b], add=True)

    load(0, 0)
    load(1, 1)
    wait_load(0)
    pltpu.async_copy(g_hbm.at[idx_v.at[0, 0]], rows_v.at[0], gsem[0])
    for i in range(NB):
      emit(i, i, wait_s=(i >= 2), do_next=True, guard_load=False)
    def outer(k, carry):
      for b in range(NB):
        emit(k * NB + b, b, wait_s=True, do_next=True, guard_load=True)
      return carry
    lax.fori_loop(1, NCHUNK // NB, outer, 0)
    emit(NCHUNK - 1, (NCHUNK - 1) % NB, wait_s=True, do_next=False,
         guard_load=False)
    for b, n_drain in enumerate(_drain_counts()):
      for _ in range(n_drain):
        wait_scatter(b)
    plsc.subcore_barrier()
    for k in range(KR):
      pltpu.sync_copy(agg_s.at[pl.ds(s * RPT + k * CH, CH)], stage_v)
      pltpu.sync_copy(stage_v, pagg_hbm.at[c, pl.ds(s * RPT + k * CH, CH)])

  return pl.kernel(body, out_type=out_type, mesh=mesh, scratch_types=scratch)


def _make_deg():
  mesh = plsc.VectorSubcoreMesh(core_axis_name="c", subcore_axis_name="s",
                                num_cores=NC, num_subcores=NS)
  out_type = jax.ShapeDtypeStruct((NC, NP, HID), jnp.float32)
  scratch = [
      pltpu.VMEM((NB, 2, CH), jnp.int32),
      pltpu.VMEM((CH, HID), jnp.float32),
      pltpu.VMEM_SHARED((NP, HID), jnp.float32),
  ] + [pltpu.SemaphoreType.DMA] * (2 * NB)
  KR = RPT // CH

  def body(ei_hbm, pdeg_hbm, idx_v, rows_v, deg_s, *sems):
    lsem = sems[0:NB]
    ssem = sems[NB:2 * NB]
    c = lax.axis_index("c")
    s = lax.axis_index("s")
    zv = jnp.zeros((16,), jnp.float32)
    def zrow(i, carry):
      for j in range(HID // 16):
        rows_v[i, pl.ds(j * 16, 16)] = zv
      return carry
    lax.fori_loop(0, CH, zrow, 0)
    for k in range(KR):
      pltpu.sync_copy(rows_v, deg_s.at[pl.ds(s * RPT + k * CH, CH)])
    ov = jnp.ones((16,), jnp.float32)
    def orow(i, carry):
      for j in range(HID // 16):
        rows_v[i, pl.ds(j * 16, 16)] = ov
      return carry
    lax.fori_loop(0, CH, orow, 0)
    plsc.subcore_barrier()

    base = (s * NC + c) * NCHUNK

    def load(i, b):
      pltpu.async_copy(ei_hbm.at[:, base + i, :], idx_v.at[b], lsem[b])

    def wait_load(b):
      pltpu.make_async_copy(ei_hbm.at[:, base, :], idx_v.at[b], lsem[b]).wait()

    def wait_scatter(b):
      pltpu.make_async_copy(pdeg_hbm.at[0, pl.ds(0, CH)], rows_v, ssem[b]).wait()

    def emit(i, b, wait_s, do_next, guard_load):
      bn2 = (b + 2) % NB
      if wait_s:
        wait_scatter(bn2)
      if do_next:
        if guard_load:
          @pl.when(i + 2 < NCHUNK)
          def _():
            load(i + 2, bn2)
        else:
          load(i + 2, bn2)
      wait_load(b)
      pltpu.async_copy(rows_v, deg_s.at[idx_v.at[b, 1]], ssem[b], add=True)

    load(0, 0)
    load(1, 1)
    for i in range(NB):
      emit(i, i, wait_s=(i >= 2), do_next=True, guard_load=False)
    def outer(k, carry):
      for b in range(NB):
        emit(k * NB + b, b, wait_s=True, do_next=True, guard_load=True)
      return carry
    lax.fori_loop(1, NCHUNK // NB, outer, 0)
    emit(NCHUNK - 1, (NCHUNK - 1) % NB, wait_s=True, do_next=False,
         guard_load=False)
    for b, n_drain in enumerate(_drain_counts()):
      for _ in range(n_drain):
        wait_scatter(b)
    plsc.subcore_barrier()
    for k in range(KR):
      pltpu.sync_copy(deg_s.at[pl.ds(s * RPT + k * CH, CH)], rows_v)
      pltpu.sync_copy(rows_v, pdeg_hbm.at[c, pl.ds(s * RPT + k * CH, CH)])

  return pl.kernel(body, out_type=out_type, mesh=mesh, scratch_types=scratch)


_make_spmm_cached = functools.lru_cache(maxsize=None)(_make_spmm)
_make_deg_cached = functools.lru_cache(maxsize=None)(_make_deg)


BN = 1000


def _mm0_body(x_ref, wc_ref, b_ref, s_ref, g_ref):
  r = jnp.dot(x_ref[...], wc_ref[...], preferred_element_type=jnp.float32)
  s_ref[...] = r[:, :HID] + b_ref[...]
  g_ref[...] = r[:, HID:]


def _comb_mm_body(dout, s_in_ref, p_ref, pd_ref, wc_ref, b_ref,
                  s_ref, g_ref):
  agg = p_ref[0] + p_ref[1]
  deg = pd_ref[0, :, 0:1] + pd_ref[1, :, 0:1]
  inv = 1.0 / jnp.maximum(deg, 1.0)
  h = jnp.maximum(s_in_ref[...] + agg * inv, 0.0)
  r = jnp.dot(h, wc_ref[...], preferred_element_type=jnp.float32)
  s_ref[...] = r[:, :dout] + b_ref[...]
  g_ref[...] = r[:, dout:]


def _final_body(s_in_ref, p_ref, pd_ref, out_ref):
  agg = p_ref[0] + p_ref[1]
  deg = pd_ref[0, :, 0:1] + pd_ref[1, :, 0:1]
  inv = 1.0 / jnp.maximum(deg, 1.0)
  out_ref[...] = s_in_ref[...] + agg * inv


def _mm0(x, wc, b):
  return pl.pallas_call(
      _mm0_body,
      grid=(N // BN,),
      in_specs=[
          pl.BlockSpec((BN, F), lambda i: (i, 0)),
          pl.BlockSpec((F, 2 * HID), lambda i: (0, 0)),
          pl.BlockSpec((1, HID), lambda i: (0, 0)),
      ],
      out_specs=[
          pl.BlockSpec((BN, HID), lambda i: (i, 0)),
          pl.BlockSpec((BN, HID), lambda i: (i, 0)),
      ],
      out_shape=[
          jax.ShapeDtypeStruct((N, HID), jnp.float32),
          jax.ShapeDtypeStruct((N, HID), jnp.float32),
      ],
  )(x, wc, b)


def _comb_mm(s_in, pagg, pdeg, wc, b, din, dout):
  return pl.pallas_call(
      functools.partial(_comb_mm_body, dout),
      grid=(N // BN,),
      in_specs=[
          pl.BlockSpec((BN, din), lambda i: (i, 0)),
          pl.BlockSpec((NC, BN, din), lambda i: (0, i, 0)),
          pl.BlockSpec((NC, BN, HID), lambda i: (0, i, 0)),
          pl.BlockSpec((din, 2 * dout), lambda i: (0, 0)),
          pl.BlockSpec((1, dout), lambda i: (0, 0)),
      ],
      out_specs=[
          pl.BlockSpec((BN, dout), lambda i: (i, 0)),
          pl.BlockSpec((BN, dout), lambda i: (i, 0)),
      ],
      out_shape=[
          jax.ShapeDtypeStruct((N, dout), jnp.float32),
          jax.ShapeDtypeStruct((N, dout), jnp.float32),
      ],
  )(s_in, pagg, pdeg, wc, b)


def _final(s_in, pagg, pdeg):
  return pl.pallas_call(
      _final_body,
      grid=(N // BN,),
      in_specs=[
          pl.BlockSpec((BN, CP), lambda i: (i, 0)),
          pl.BlockSpec((NC, BN, CP), lambda i: (0, i, 0)),
          pl.BlockSpec((NC, BN, HID), lambda i: (0, i, 0)),
      ],
      out_specs=pl.BlockSpec((BN, CP), lambda i: (i, 0)),
      out_shape=jax.ShapeDtypeStruct((N, CP), jnp.float32),
  )(s_in, pagg, pdeg)


def kernel(x, edge_index, W_self_0, W_neigh_0, b_0,
           W_self_1, W_neigh_1, b_1, W_self_2, W_neigh_2, b_2):
  ei3 = edge_index.reshape(2, NW * NCHUNK, CH)

  pdeg = _make_deg_cached()(ei3)

  wc0 = jnp.concatenate([W_self_0, W_neigh_0], axis=1)
  s0, g0 = _mm0(x, wc0, b_0.reshape(1, HID))
  pagg0 = _make_spmm_cached(HID)(ei3, g0)

  wc1 = jnp.concatenate([W_self_1, W_neigh_1], axis=1)
  s1, g1 = _comb_mm(s0, pagg0, pdeg, wc1, b_1.reshape(1, HID), HID, HID)
  pagg1 = _make_spmm_cached(HID)(ei3, g1)

  wc2 = jnp.concatenate([
      jnp.pad(W_self_2, ((0, 0), (0, CP - C))),
      jnp.pad(W_neigh_2, ((0, 0), (0, CP - C))),
  ], axis=1)
  b2p = jnp.pad(b_2, (0, CP - C)).reshape(1, CP)
  s2, g2 = _comb_mm(s1, pagg1, pdeg, wc2, b2p, HID, CP)
  pagg2 = _make_spmm_cached(CP)(ei3, g2)

  out = _final(s2, pagg2, pdeg)
  return out[:, :C]


# speedup vs baseline: 12.4118x; 1.0007x over previous
import functools

import jax
import jax.numpy as jnp
from jax import lax
from jax.experimental import pallas as pl
from jax.experimental.pallas import tpu as pltpu
from jax.experimental.pallas import tpu_sc as plsc

N = 10000
E = 320000
F = 128
HID = 128
C = 47
CP = 128

NC = 2
NS = 16
NW = NC * NS
EPT = E // NW
CH = 80
NCHUNK = EPT // CH
NP = 10240
RPT = NP // NS
NB = 4


def _drain_counts():
  issued = [0] * NB
  waited = [0] * NB
  for i in range(NCHUNK):
    issued[i % NB] += 1
    if i >= 2:
      waited[(i + 2) % NB] += 1
  return [issued[j] - waited[j] for j in range(NB)]


def _make_spmm(D):
  mesh = plsc.VectorSubcoreMesh(core_axis_name="c", subcore_axis_name="s",
                                num_cores=NC, num_subcores=NS)
  out_type = jax.ShapeDtypeStruct((NC, NP, D), jnp.float32)
  scratch = [
      pltpu.VMEM((NB, 2, CH), jnp.int32),
      pltpu.VMEM((NB, CH, D), jnp.float32),
      pltpu.VMEM_SHARED((NP, D), jnp.float32),
  ] + [pltpu.SemaphoreType.DMA] * (3 * NB)
  KR = RPT // CH

  def body(ei_hbm, g_hbm, pagg_hbm, idx_v, rows_v, agg_s, *sems):
    stage_v = rows_v.at[0]
    lsem = sems[0:NB]
    gsem = sems[NB:2 * NB]
    ssem = sems[2 * NB:3 * NB]
    c = lax.axis_index("c")
    s = lax.axis_index("s")
    zv = jnp.zeros((16,), jnp.float32)
    def zrow(i, carry):
      for j in range(D // 16):
        stage_v[i, pl.ds(j * 16, 16)] = zv
      return carry
    lax.fori_loop(0, CH, zrow, 0)
    for k in range(KR):
      pltpu.sync_copy(stage_v, agg_s.at[pl.ds(s * RPT + k * CH, CH)])
    plsc.subcore_barrier()

    base = (s * NC + c) * NCHUNK

    def load(i, b):
      pltpu.async_copy(ei_hbm.at[:, base + i, :], idx_v.at[b], lsem[b])

    def wait_load(b):
      pltpu.make_async_copy(ei_hbm.at[:, base, :], idx_v.at[b], lsem[b]).wait()

    def wait_gather(b):
      pltpu.make_async_copy(g_hbm.at[pl.ds(0, CH)], rows_v.at[b], gsem[b]).wait()

    def wait_scatter(b):
      pltpu.make_async_copy(g_hbm.at[pl.ds(0, CH)], rows_v.at[b], ssem[b]).wait()

    def emit(i, b, wait_s, do_next, guard_load):
      bn = (b + 1) % NB
      bn2 = (b + 2) % NB
      if wait_s:
        wait_scatter(bn2)
      if do_next:
        wait_load(bn)
        pltpu.async_copy(g_hbm.at[idx_v.at[bn, 0]], rows_v.at[bn], gsem[bn])
        if guard_load:
          @pl.when(i + 2 < NCHUNK)
          def _():
            load(i + 2, bn2)
        else:
          load(i + 2, bn2)
      wait_gather(b)
      pltpu.async_copy(rows_v.at[b], agg_s.at[idx_v.at[b, 1]], ssem[b], add=True)

    load(0, 0)
    load(1, 1)
    wait_load(0)
    pltpu.async_copy(g_hbm.at[idx_v.at[0, 0]], rows_v.at[0], gsem[0])
    for i in range(NB):
      emit(i, i, wait_s=(i >= 2), do_next=True, guard_load=False)
    def outer(k, carry):
      for b in range(NB):
        emit(k * NB + b, b, wait_s=True, do_next=True, guard_load=True)
      return carry
    lax.fori_loop(1, NCHUNK // NB, outer, 0)
    emit(NCHUNK - 1, (NCHUNK - 1) % NB, wait_s=True, do_next=False,
         guard_load=False)
    for b, n_drain in enumerate(_drain_counts()):
      for _ in range(n_drain):
        wait_scatter(b)
    plsc.subcore_barrier()
    for k in range(KR):
      pltpu.sync_copy(agg_s.at[pl.ds(s * RPT + k * CH, CH)], stage_v)
      pltpu.sync_copy(stage_v, pagg_hbm.at[c, pl.ds(s * RPT + k * CH, CH)])

  return pl.kernel(body, out_type=out_type, mesh=mesh, scratch_types=scratch)


def _make_deg():
  mesh = plsc.VectorSubcoreMesh(core_axis_name="c", subcore_axis_name="s",
                                num_cores=NC, num_subcores=NS)
  out_type = jax.ShapeDtypeStruct((NC, NP, HID), jnp.float32)
  scratch = [
      pltpu.VMEM((NB, 2, CH), jnp.int32),
      pltpu.VMEM((CH, HID), jnp.float32),
      pltpu.VMEM_SHARED((NP, HID), jnp.float32),
  ] + [pltpu.SemaphoreType.DMA] * (2 * NB)
  KR = RPT // CH

  def body(ei_hbm, pdeg_hbm, idx_v, rows_v, deg_s, *sems):
    lsem = sems[0:NB]
    ssem = sems[NB:2 * NB]
    c = lax.axis_index("c")
    s = lax.axis_index("s")
    zv = jnp.zeros((16,), jnp.float32)
    def zrow(i, carry):
      for j in range(HID // 16):
        rows_v[i, pl.ds(j * 16, 16)] = zv
      return carry
    lax.fori_loop(0, CH, zrow, 0)
    for k in range(KR):
      pltpu.sync_copy(rows_v, deg_s.at[pl.ds(s * RPT + k * CH, CH)])
    ov = jnp.ones((16,), jnp.float32)
    def orow(i, carry):
      for j in range(HID // 16):
        rows_v[i, pl.ds(j * 16, 16)] = ov
      return carry
    lax.fori_loop(0, CH, orow, 0)
    plsc.subcore_barrier()

    base = (s * NC + c) * NCHUNK

    def load(i, b):
      pltpu.async_copy(ei_hbm.at[:, base + i, :], idx_v.at[b], lsem[b])

    def wait_load(b):
      pltpu.make_async_copy(ei_hbm.at[:, base, :], idx_v.at[b], lsem[b]).wait()

    def wait_scatter(b):
      pltpu.make_async_copy(pdeg_hbm.at[0, pl.ds(0, CH)], rows_v, ssem[b]).wait()

    def emit(i, b, wait_s, do_next, guard_load):
      bn2 = (b + 2) % NB
      if wait_s:
        wait_scatter(bn2)
      if do_next:
        if guard_load:
          @pl.when(i + 2 < NCHUNK)
          def _():
            load(i + 2, bn2)
        else:
          load(i + 2, bn2)
      wait_load(b)
      pltpu.async_copy(rows_v, deg_s.at[idx_v.at[b, 1]], ssem[b], add=True)

    load(0, 0)
    load(1, 1)
    for i in range(NB):
      emit(i, i, wait_s=(i >= 2), do_next=True, guard_load=False)
    def outer(k, carry):
      for b in range(NB):
        emit(k * NB + b, b, wait_s=True, do_next=True, guard_load=True)
      return carry
    lax.fori_loop(1, NCHUNK // NB, outer, 0)
    emit(NCHUNK - 1, (NCHUNK - 1) % NB, wait_s=True, do_next=False,
         guard_load=False)
    for b, n_drain in enumerate(_drain_counts()):
      for _ in range(n_drain):
        wait_scatter(b)
    plsc.subcore_barrier()
    for k in range(KR):
      pltpu.sync_copy(deg_s.at[pl.ds(s * RPT + k * CH, CH)], rows_v)
      pltpu.sync_copy(rows_v, pdeg_hbm.at[c, pl.ds(s * RPT + k * CH, CH)])

  return pl.kernel(body, out_type=out_type, mesh=mesh, scratch_types=scratch)


_make_spmm_cached = functools.lru_cache(maxsize=None)(_make_spmm)
_make_deg_cached = functools.lru_cache(maxsize=None)(_make_deg)


BN = 1000


def _mm0_body(x_ref, wc_ref, b_ref, s_ref, g_ref):
  r = jnp.dot(x_ref[...].astype(jnp.bfloat16), wc_ref[...].astype(jnp.bfloat16),
              preferred_element_type=jnp.float32)
  s_ref[...] = r[:, :HID] + b_ref[...]
  g_ref[...] = r[:, HID:]


def _comb_mm_body(dout, s_in_ref, p_ref, pd_ref, wc_ref, b_ref,
                  s_ref, g_ref):
  agg = p_ref[0] + p_ref[1]
  deg = pd_ref[0, :, 0:1] + pd_ref[1, :, 0:1]
  inv = 1.0 / jnp.maximum(deg, 1.0)
  h = jnp.maximum(s_in_ref[...] + agg * inv, 0.0)
  r = jnp.dot(h.astype(jnp.bfloat16), wc_ref[...].astype(jnp.bfloat16),
              preferred_element_type=jnp.float32)
  s_ref[...] = r[:, :dout] + b_ref[...]
  g_ref[...] = r[:, dout:]


def _final_body(s_in_ref, p_ref, pd_ref, out_ref):
  agg = p_ref[0] + p_ref[1]
  deg = pd_ref[0, :, 0:1] + pd_ref[1, :, 0:1]
  inv = 1.0 / jnp.maximum(deg, 1.0)
  out_ref[...] = s_in_ref[...] + agg * inv


def _mm0(x, wc, b):
  return pl.pallas_call(
      _mm0_body,
      grid=(N // BN,),
      in_specs=[
          pl.BlockSpec((BN, F), lambda i: (i, 0)),
          pl.BlockSpec((F, 2 * HID), lambda i: (0, 0)),
          pl.BlockSpec((1, HID), lambda i: (0, 0)),
      ],
      out_specs=[
          pl.BlockSpec((BN, HID), lambda i: (i, 0)),
          pl.BlockSpec((BN, HID), lambda i: (i, 0)),
      ],
      out_shape=[
          jax.ShapeDtypeStruct((N, HID), jnp.float32),
          jax.ShapeDtypeStruct((N, HID), jnp.float32),
      ],
  )(x, wc, b)


def _comb_mm(s_in, pagg, pdeg, wc, b, din, dout):
  return pl.pallas_call(
      functools.partial(_comb_mm_body, dout),
      grid=(N // BN,),
      in_specs=[
          pl.BlockSpec((BN, din), lambda i: (i, 0)),
          pl.BlockSpec((NC, BN, din), lambda i: (0, i, 0)),
          pl.BlockSpec((NC, BN, HID), lambda i: (0, i, 0)),
          pl.BlockSpec((din, 2 * dout), lambda i: (0, 0)),
          pl.BlockSpec((1, dout), lambda i: (0, 0)),
      ],
      out_specs=[
          pl.BlockSpec((BN, dout), lambda i: (i, 0)),
          pl.BlockSpec((BN, dout), lambda i: (i, 0)),
      ],
      out_shape=[
          jax.ShapeDtypeStruct((N, dout), jnp.float32),
          jax.ShapeDtypeStruct((N, dout), jnp.float32),
      ],
  )(s_in, pagg, pdeg, wc, b)


def _final(s_in, pagg, pdeg):
  return pl.pallas_call(
      _final_body,
      grid=(N // BN,),
      in_specs=[
          pl.BlockSpec((BN, CP), lambda i: (i, 0)),
          pl.BlockSpec((NC, BN, CP), lambda i: (0, i, 0)),
          pl.BlockSpec((NC, BN, HID), lambda i: (0, i, 0)),
      ],
      out_specs=pl.BlockSpec((BN, CP), lambda i: (i, 0)),
      out_shape=jax.ShapeDtypeStruct((N, CP), jnp.float32),
  )(s_in, pagg, pdeg)


def kernel(x, edge_index, W_self_0, W_neigh_0, b_0,
           W_self_1, W_neigh_1, b_1, W_self_2, W_neigh_2, b_2):
  ei3 = edge_index.reshape(2, NW * NCHUNK, CH)

  pdeg = _make_deg_cached()(ei3)

  wc0 = jnp.concatenate([W_self_0, W_neigh_0], axis=1)
  s0, g0 = _mm0(x, wc0, b_0.reshape(1, HID))
  pagg0 = _make_spmm_cached(HID)(ei3, g0)

  wc1 = jnp.concatenate([W_self_1, W_neigh_1], axis=1)
  s1, g1 = _comb_mm(s0, pagg0, pdeg, wc1, b_1.reshape(1, HID), HID, HID)
  pagg1 = _make_spmm_cached(HID)(ei3, g1)

  wc2 = jnp.concatenate([
      jnp.pad(W_self_2, ((0, 0), (0, CP - C))),
      jnp.pad(W_neigh_2, ((0, 0), (0, CP - C))),
  ], axis=1)
  b2p = jnp.pad(b_2, (0, CP - C)).reshape(1, CP)
  s2, g2 = _comb_mm(s1, pagg1, pdeg, wc2, b2p, HID, CP)
  pagg2 = _make_spmm_cached(CP)(ei3, g2)

  out = _final(s2, pagg2, pdeg)
  return out[:, :C]
